# baseline (speedup 1.0000x reference)
import functools

import jax
import jax.numpy as jnp
from jax import lax
from jax.experimental import pallas as pl
from jax.experimental.pallas import tpu as pltpu
from jax.experimental.pallas import tpu_sc as plsc

_DECAY = 0.9
_NC, _NS, _L = 2, 16, 16
_NW = _NC * _NS


def _wid():
    return lax.axis_index("s") * _NC + lax.axis_index("c")


def _mesh():
    return plsc.VectorSubcoreMesh(
        core_axis_name="c", subcore_axis_name="s",
        num_cores=_NC, num_subcores=_NS)


_SC_PARAMS = pltpu.CompilerParams(needs_layout_passes=False)


def _build_pos(class_s, class_t, m_pad, r_per_w):
    b = class_s.shape[0]

    def body(cls_s_hbm, cls_t_hbm, pos_s_hbm, pos_t_hbm,
             cls_s_v, cls_t_v, pos_s_v, pos_t_v):
        w = _wid()
        base_c = w * r_per_w
        pltpu.sync_copy(cls_s_hbm, cls_s_v)
        pltpu.sync_copy(cls_t_hbm, cls_t_v)

        neg1 = jnp.full((_L,), -1, jnp.int32)

        def init_body(r, carry):
            pos_s_v[pl.ds(r * _L, _L)] = neg1
            pos_t_v[pl.ds(r * _L, _L)] = neg1
            return carry

        lax.fori_loop(0, r_per_w // _L, init_body, 0)

        iota = lax.broadcasted_iota(jnp.int32, (_L,), 0)

        def scan_body(i, carry):
            ivec = iota + i * _L
            c_s = cls_s_v[pl.ds(i * _L, _L)] - base_c
            m_s = (c_s >= 0) & (c_s < r_per_w)
            plsc.store_scatter(pos_s_v, [jnp.where(m_s, c_s, 0)], ivec,
                               mask=m_s)
            c_t = cls_t_v[pl.ds(i * _L, _L)] - base_c
            m_t = (c_t >= 0) & (c_t < r_per_w)
            plsc.store_scatter(pos_t_v, [jnp.where(m_t, c_t, 0)], ivec,
                               mask=m_t)
            return carry

        lax.fori_loop(0, b // _L, scan_body, 0)

        pltpu.sync_copy(pos_s_v, pos_s_hbm.at[pl.ds(base_c, r_per_w)])
        pltpu.sync_copy(pos_t_v, pos_t_hbm.at[pl.ds(base_c, r_per_w)])

    run = pl.kernel(
        body,
        out_type=[jax.ShapeDtypeStruct((m_pad,), jnp.int32),
                  jax.ShapeDtypeStruct((m_pad,), jnp.int32)],
        mesh=_mesh(),
        scratch_types=[pltpu.VMEM((b,), jnp.int32),
                       pltpu.VMEM((b,), jnp.int32),
                       pltpu.VMEM((r_per_w,), jnp.int32),
                       pltpu.VMEM((r_per_w,), jnp.int32)],
        compiler_params=_SC_PARAMS,
    )
    return run(class_s, class_t)


def _corrections(s_f, t_f, f_s, f_t, class_s, class_t, pos_s, pos_t):
    m, d = s_f.shape
    b = class_s.shape[0]
    ch = b // _NW
    g = 128
    nq = ch // g

    def side(w, acc, is_s_side, cls_hbm, flin_hbm, ft_hbm,
             pos_s_hbm, pos_t_hbm, s_hbm, t_hbm,
             cls_v, ps_v, pt_v, ptc_v, sv_ref, tv_ref, f1_ref, f2_ref):
        ibase = w * ch
        pltpu.sync_copy(cls_hbm.at[pl.ds(ibase, ch)], cls_v)
        iota = lax.broadcasted_iota(jnp.int32, (_L,), 0)
        for q in range(nq):
            idx_ref = cls_v.at[pl.ds(q * g, g)]
            pltpu.sync_copy(pos_s_hbm.at[idx_ref], ps_v)
            pltpu.sync_copy(pos_t_hbm.at[idx_ref], pt_v)
            pltpu.sync_copy(s_hbm.at[idx_ref], sv_ref)
            pltpu.sync_copy(t_hbm.at[idx_ref], tv_ref)
            pltpu.sync_copy(flin_hbm.at[pl.ds(ibase + q * g, g)], f1_ref)

            if is_s_side:
                def clamp_body(r, carry):
                    ptc_v[pl.ds(r * _L, _L)] = jnp.maximum(
                        pt_v[pl.ds(r * _L, _L)], 0)
                    return carry
                lax.fori_loop(0, g // _L, clamp_body, 0)
                pltpu.sync_copy(ft_hbm.at[ptc_v], f2_ref)

            def grp_body(gg, acc_in):
                rows16 = gg * _L + iota
                pt16 = pt_v[pl.ds(gg * _L, _L)]
                ps16 = ps_v[pl.ds(gg * _L, _L)]
                bidx16 = ibase + q * g + gg * _L + iota
                if is_s_side:
                    win = ps16 == bidx16
                    has_t = pt16 >= 0
                else:
                    win = (pt16 == bidx16) & (ps16 < 0)

                def col_body(cb, carry):
                    acc_n, acc_o = carry
                    for u in range(4):
                        c16 = jnp.full((_L,), cb * 4 + u, jnp.int32)
                        sv = plsc.load_gather(sv_ref, [rows16, c16])
                        tv = plsc.load_gather(tv_ref, [rows16, c16])
                        f1 = plsc.load_gather(f1_ref, [rows16, c16])
                        dv = sv - tv
                        if is_s_side:
                            f2 = plsc.load_gather(f2_ref, [rows16, c16])
                            ev = f1 - jnp.where(has_t, f2, tv)
                        else:
                            ev = sv - f1
                        acc_n = acc_n + jnp.abs(0.9 * dv + 0.1 * ev)
                        acc_o = acc_o + jnp.abs(dv)
                    return acc_n, acc_o

                zeros = jnp.zeros((_L,), jnp.float32)
                acc_n, acc_o = lax.fori_loop(0, d // 4, col_body,
                                             (zeros, zeros))
                return acc_in + jnp.where(win, acc_n - acc_o, 0.0)

            acc = lax.fori_loop(0, g // _L, grp_body, acc)
        return acc

    def body(s_hbm, t_hbm, fs_hbm, ft_hbm, cls_s_hbm, cls_t_hbm,
             pos_s_hbm, pos_t_hbm, out_hbm,
             cls_v, ps_v, pt_v, ptc_v, sv_ref, tv_ref, f1_ref, f2_ref,
             acc_v):
        w = _wid()
        acc = jnp.zeros((_L,), jnp.float32)
        acc = side(w, acc, True, cls_s_hbm, fs_hbm, ft_hbm,
                   pos_s_hbm, pos_t_hbm, s_hbm, t_hbm,
                   cls_v, ps_v, pt_v, ptc_v, sv_ref, tv_ref, f1_ref, f2_ref)
        acc = side(w, acc, False, cls_t_hbm, ft_hbm, ft_hbm,
                   pos_s_hbm, pos_t_hbm, s_hbm, t_hbm,
                   cls_v, ps_v, pt_v, ptc_v, sv_ref, tv_ref, f1_ref, f2_ref)
        acc_v[...] = acc
        pltpu.sync_copy(acc_v, out_hbm.at[w])

    run = pl.kernel(
        body,
        out_type=jax.ShapeDtypeStruct((_NW, _L), jnp.float32),
        mesh=_mesh(),
        scratch_types=[pltpu.VMEM((ch,), jnp.int32),
                       pltpu.VMEM((g,), jnp.int32),
                       pltpu.VMEM((g,), jnp.int32),
                       pltpu.VMEM((g,), jnp.int32),
                       pltpu.VMEM((g, d), jnp.float32),
                       pltpu.VMEM((g, d), jnp.float32),
                       pltpu.VMEM((g, d), jnp.float32),
                       pltpu.VMEM((g, d), jnp.float32),
                       pltpu.VMEM((_L,), jnp.float32)],
        compiler_params=_SC_PARAMS,
    )
    return run(s_f, t_f, f_s, f_t, class_s, class_t, pos_s, pos_t)


def _dense_base(s_f, t_f, corr, blk, grid):
    m, d = s_f.shape

    def body(s_ref, t_ref, corr_ref, out_ref):
        i = pl.program_id(0)

        @pl.when(i == 0)
        def _():
            out_ref[0, 0] = 0.0

        out_ref[0, 0] += jnp.sum(jnp.abs(s_ref[...] - t_ref[...]))

        @pl.when(i == grid - 1)
        def _():
            out_ref[0, 0] = (out_ref[0, 0] + jnp.sum(corr_ref[...])) / m

    return pl.pallas_call(
        body,
        grid=(grid,),
        in_specs=[pl.BlockSpec((blk, d), lambda i: (i, 0)),
                  pl.BlockSpec((blk, d), lambda i: (i, 0)),
                  pl.BlockSpec((_NW, _L), lambda i: (0, 0))],
        out_specs=pl.BlockSpec((1, 1), lambda i: (0, 0),
                               memory_space=pltpu.SMEM),
        out_shape=jax.ShapeDtypeStruct((1, 1), jnp.float32),
    )(s_f, t_f, corr)


def kernel(s_feature, t_feature, feature_s, feature_t, class_s, class_t):
    m, d = s_feature.shape
    r_per_w = (m + _NW - 1) // _NW
    r_per_w = (r_per_w + _L - 1) // _L * _L
    m_pad = r_per_w * _NW

    pos_s, pos_t = _build_pos(class_s, class_t, m_pad, r_per_w)
    corr = _corrections(s_feature, t_feature, feature_s, feature_t,
                        class_s, class_t, pos_s, pos_t)
    blk = 5000 if m % 5000 == 0 else m
    out = _dense_base(s_feature, t_feature, corr, blk, m // blk)
    return out[0, 0]


# speedup vs baseline: 1.1862x; 1.1862x over previous
import functools

import jax
import jax.numpy as jnp
from jax import lax
from jax.experimental import pallas as pl
from jax.experimental.pallas import tpu as pltpu
from jax.experimental.pallas import tpu_sc as plsc

_DECAY = 0.9
_NC, _NS, _L = 2, 16, 16
_NW = _NC * _NS


def _wid():
    return lax.axis_index("s") * _NC + lax.axis_index("c")


def _mesh():
    return plsc.VectorSubcoreMesh(
        core_axis_name="c", subcore_axis_name="s",
        num_cores=_NC, num_subcores=_NS)


_SC_PARAMS = pltpu.CompilerParams(needs_layout_passes=False)


def _build_pos(class_s, class_t, m_pad, r_per_w):
    b = class_s.shape[0]

    def body(cls_s_hbm, cls_t_hbm, pos_s_hbm, pos_t_hbm,
             cls_s_v, cls_t_v, pos_s_v, pos_t_v):
        w = _wid()
        base_c = w * r_per_w
        pltpu.sync_copy(cls_s_hbm, cls_s_v)
        pltpu.sync_copy(cls_t_hbm, cls_t_v)

        neg1 = jnp.full((_L,), -1, jnp.int32)

        def init_body(r, carry):
            pos_s_v[pl.ds(r * _L, _L)] = neg1
            pos_t_v[pl.ds(r * _L, _L)] = neg1
            return carry

        lax.fori_loop(0, r_per_w // _L, init_body, 0)

        iota = lax.broadcasted_iota(jnp.int32, (_L,), 0)

        def scan_body(i, carry):
            ivec = iota + i * _L
            c_s = cls_s_v[pl.ds(i * _L, _L)] - base_c
            m_s = (c_s >= 0) & (c_s < r_per_w)
            plsc.store_scatter(pos_s_v, [jnp.where(m_s, c_s, 0)], ivec,
                               mask=m_s)
            c_t = cls_t_v[pl.ds(i * _L, _L)] - base_c
            m_t = (c_t >= 0) & (c_t < r_per_w)
            plsc.store_scatter(pos_t_v, [jnp.where(m_t, c_t, 0)], ivec,
                               mask=m_t)
            return carry

        lax.fori_loop(0, b // _L, scan_body, 0)

        pltpu.sync_copy(pos_s_v, pos_s_hbm.at[pl.ds(base_c, r_per_w)])
        pltpu.sync_copy(pos_t_v, pos_t_hbm.at[pl.ds(base_c, r_per_w)])

    run = pl.kernel(
        body,
        out_type=[jax.ShapeDtypeStruct((m_pad,), jnp.int32),
                  jax.ShapeDtypeStruct((m_pad,), jnp.int32)],
        mesh=_mesh(),
        scratch_types=[pltpu.VMEM((b,), jnp.int32),
                       pltpu.VMEM((b,), jnp.int32),
                       pltpu.VMEM((r_per_w,), jnp.int32),
                       pltpu.VMEM((r_per_w,), jnp.int32)],
        compiler_params=_SC_PARAMS,
    )
    return run(class_s, class_t)


def _corrections(s_f, t_f, f_s, f_t, class_s, class_t, pos_s, pos_t):
    m, d = s_f.shape
    b = class_s.shape[0]
    ch = b // _NW
    g = 128
    nq = ch // g

    def side(w, acc, is_s_side, cls_hbm, flin_hbm, ft_hbm,
             pos_s_hbm, pos_t_hbm, s_hbm, t_hbm,
             cls_v, ps_v, pt_v, ptc_v, sv_ref, tv_ref, f1_ref, f2_ref):
        ibase = w * ch
        pltpu.sync_copy(cls_hbm.at[pl.ds(ibase, ch)], cls_v)
        iota = lax.broadcasted_iota(jnp.int32, (_L,), 0)
        for q in range(nq):
            idx_ref = cls_v.at[pl.ds(q * g, g)]
            pltpu.sync_copy(pos_s_hbm.at[idx_ref], ps_v)
            pltpu.sync_copy(pos_t_hbm.at[idx_ref], pt_v)
            pltpu.sync_copy(s_hbm.at[idx_ref], sv_ref)
            pltpu.sync_copy(t_hbm.at[idx_ref], tv_ref)
            pltpu.sync_copy(flin_hbm.at[pl.ds(ibase + q * g, g)], f1_ref)

            if is_s_side:
                def clamp_body(r, carry):
                    ptc_v[pl.ds(r * _L, _L)] = jnp.maximum(
                        pt_v[pl.ds(r * _L, _L)], 0)
                    return carry
                lax.fori_loop(0, g // _L, clamp_body, 0)
                pltpu.sync_copy(ft_hbm.at[ptc_v], f2_ref)

            def grp_body(gg, acc_in):
                rows16 = gg * _L + iota
                pt16 = pt_v[pl.ds(gg * _L, _L)]
                ps16 = ps_v[pl.ds(gg * _L, _L)]
                bidx16 = ibase + q * g + gg * _L + iota
                if is_s_side:
                    win = ps16 == bidx16
                    has_t = pt16 >= 0
                else:
                    win = (pt16 == bidx16) & (ps16 < 0)

                def col_body(cb, carry):
                    acc_n, acc_o = carry
                    for u in range(4):
                        c16 = (iota + (cb * 4 + u)) & (d - 1)
                        sv = plsc.load_gather(sv_ref, [rows16, c16])
                        tv = plsc.load_gather(tv_ref, [rows16, c16])
                        f1 = plsc.load_gather(f1_ref, [rows16, c16])
                        dv = sv - tv
                        if is_s_side:
                            f2 = plsc.load_gather(f2_ref, [rows16, c16])
                            ev = f1 - jnp.where(has_t, f2, tv)
                        else:
                            ev = sv - f1
                        acc_n = acc_n + jnp.abs(0.9 * dv + 0.1 * ev)
                        acc_o = acc_o + jnp.abs(dv)
                    return acc_n, acc_o

                zeros = jnp.zeros((_L,), jnp.float32)
                acc_n, acc_o = lax.fori_loop(0, d // 4, col_body,
                                             (zeros, zeros))
                return acc_in + jnp.where(win, acc_n - acc_o, 0.0)

            acc = lax.fori_loop(0, g // _L, grp_body, acc)
        return acc

    def body(s_hbm, t_hbm, fs_hbm, ft_hbm, cls_s_hbm, cls_t_hbm,
             pos_s_hbm, pos_t_hbm, out_hbm,
             cls_v, ps_v, pt_v, ptc_v, sv_ref, tv_ref, f1_ref, f2_ref,
             acc_v):
        w = _wid()
        acc = jnp.zeros((_L,), jnp.float32)
        acc = side(w, acc, True, cls_s_hbm, fs_hbm, ft_hbm,
                   pos_s_hbm, pos_t_hbm, s_hbm, t_hbm,
                   cls_v, ps_v, pt_v, ptc_v, sv_ref, tv_ref, f1_ref, f2_ref)
        acc = side(w, acc, False, cls_t_hbm, ft_hbm, ft_hbm,
                   pos_s_hbm, pos_t_hbm, s_hbm, t_hbm,
                   cls_v, ps_v, pt_v, ptc_v, sv_ref, tv_ref, f1_ref, f2_ref)
        acc_v[...] = acc
        pltpu.sync_copy(acc_v, out_hbm.at[w])

    run = pl.kernel(
        body,
        out_type=jax.ShapeDtypeStruct((_NW, _L), jnp.float32),
        mesh=_mesh(),
        scratch_types=[pltpu.VMEM((ch,), jnp.int32),
                       pltpu.VMEM((g,), jnp.int32),
                       pltpu.VMEM((g,), jnp.int32),
                       pltpu.VMEM((g,), jnp.int32),
                       pltpu.VMEM((g, d), jnp.float32),
                       pltpu.VMEM((g, d), jnp.float32),
                       pltpu.VMEM((g, d), jnp.float32),
                       pltpu.VMEM((g, d), jnp.float32),
                       pltpu.VMEM((_L,), jnp.float32)],
        compiler_params=_SC_PARAMS,
    )
    return run(s_f, t_f, f_s, f_t, class_s, class_t, pos_s, pos_t)


def _dense_base(s_f, t_f, corr, blk, grid):
    m, d = s_f.shape

    def body(s_ref, t_ref, corr_ref, out_ref):
        i = pl.program_id(0)

        @pl.when(i == 0)
        def _():
            out_ref[0, 0] = 0.0

        out_ref[0, 0] += jnp.sum(jnp.abs(s_ref[...] - t_ref[...]))

        @pl.when(i == grid - 1)
        def _():
            out_ref[0, 0] = (out_ref[0, 0] + jnp.sum(corr_ref[...])) / m

    return pl.pallas_call(
        body,
        grid=(grid,),
        in_specs=[pl.BlockSpec((blk, d), lambda i: (i, 0)),
                  pl.BlockSpec((blk, d), lambda i: (i, 0)),
                  pl.BlockSpec((_NW, _L), lambda i: (0, 0))],
        out_specs=pl.BlockSpec((1, 1), lambda i: (0, 0),
                               memory_space=pltpu.SMEM),
        out_shape=jax.ShapeDtypeStruct((1, 1), jnp.float32),
    )(s_f, t_f, corr)


def kernel(s_feature, t_feature, feature_s, feature_t, class_s, class_t):
    m, d = s_feature.shape
    r_per_w = (m + _NW - 1) // _NW
    r_per_w = (r_per_w + _L - 1) // _L * _L
    m_pad = r_per_w * _NW

    pos_s, pos_t = _build_pos(class_s, class_t, m_pad, r_per_w)
    corr = _corrections(s_feature, t_feature, feature_s, feature_t,
                        class_s, class_t, pos_s, pos_t)
    blk = 5000 if m % 5000 == 0 else m
    out = _dense_base(s_feature, t_feature, corr, blk, m // blk)
    return out[0, 0]
